# baseline (speedup 1.0000x reference)
import functools

import jax
import jax.numpy as jnp
from jax import lax
from jax.experimental import pallas as pl
from jax.experimental.pallas import tpu as pltpu
from jax.experimental.pallas import tpu_sc as plsc

NUM_KMERS = 4096
DIM = 10000
DIM_PAD = 10112
MAX_POSITIONS = 1000
POS_PAD = 1024
BATCH = 1024
SEQ = 20
LANES = 16
DGROUP = 8

NUM_CORES = 2
NUM_SUBCORES = 16
NUM_WORKERS = NUM_CORES * NUM_SUBCORES
NGROUPS = DIM // DGROUP
GGROUPS = DIM_PAD // DGROUP
KMAX = (NGROUPS + NUM_WORKERS - 1) // NUM_WORKERS
EMB_TILES = NUM_KMERS // 128
POS_TILES = POS_PAD // 128
BTILES = BATCH // 128


def _t_emb_body(in_ref, out_ref):
    out_ref[...] = in_ref[...].T


def _transpose_emb(embeddings):
    return pl.pallas_call(
        _t_emb_body,
        grid=(10, 2),
        in_specs=[pl.BlockSpec((2048, 1024), lambda c, r: (r, c))],
        out_specs=pl.BlockSpec((1024, 2048), lambda c, r: (c, r)),
        out_shape=jax.ShapeDtypeStruct((DIM, NUM_KMERS), jnp.float32),
    )(embeddings)


def _sig_body(in_ref, out_ref):
    out_ref[:, :DIM] = jax.nn.sigmoid(in_ref[...])
    out_ref[:, DIM:] = jnp.ones((in_ref.shape[0], DIM_PAD - DIM), jnp.float32)


def _sigmoid_wide(position_embeddings):
    return pl.pallas_call(
        _sig_body,
        grid=(5,),
        in_specs=[pl.BlockSpec((200, DIM), lambda i: (i, 0))],
        out_specs=pl.BlockSpec((200, DIM_PAD), lambda i: (i, 0)),
        out_shape=jax.ShapeDtypeStruct((POS_PAD, DIM_PAD), jnp.float32),
    )(position_embeddings)


def _t_gate_body(in_ref, out_ref):
    out_ref[...] = in_ref[...].T


def _transpose_gate(gate_wide):
    return pl.pallas_call(
        _t_gate_body,
        grid=(10,),
        in_specs=[pl.BlockSpec((POS_PAD, 1024), lambda c: (0, c))],
        out_specs=pl.BlockSpec((1024, POS_PAD), lambda c: (c, 0)),
        out_shape=jax.ShapeDtypeStruct((DIM_PAD, POS_PAD), jnp.float32),
    )(gate_wide)


_sc_mesh = plsc.VectorSubcoreMesh(core_axis_name="c", subcore_axis_name="s")


@functools.partial(
    pl.kernel,
    mesh=_sc_mesh,
    compiler_params=pltpu.CompilerParams(use_tc_tiling_on_sc=False,
                                         needs_layout_passes=False),
    out_type=jax.ShapeDtypeStruct((SEQ, NGROUPS, BTILES, DGROUP, 128),
                                  jnp.float32),
    scratch_types=[
        pltpu.VMEM((SEQ, BATCH), jnp.int32),
        pltpu.VMEM((EMB_TILES, DGROUP, 128), jnp.float32),
        pltpu.VMEM((EMB_TILES, DGROUP, 128), jnp.float32),
        pltpu.VMEM((POS_TILES, DGROUP, 128), jnp.float32),
        pltpu.VMEM((POS_TILES, DGROUP, 128), jnp.float32),
        pltpu.VMEM((BTILES, DGROUP, 128), jnp.float32),
        pltpu.VMEM((BTILES, DGROUP, 128), jnp.float32),
        pltpu.SemaphoreType.DMA,
        pltpu.SemaphoreType.DMA,
        pltpu.SemaphoreType.DMA,
    ],
)
def _sc_gather_mul(embT_hbm, gateT_hbm, packed_hbm, out_hbm,
                   packed_v, embT_v0, embT_v1, gateT_v0, gateT_v1,
                   ob0, ob1, sem0, sem1, sem_in):
    wid = lax.axis_index("s") * NUM_CORES + lax.axis_index("c")
    pltpu.sync_copy(packed_hbm, packed_v)
    obufs = (ob0, ob1)
    sems = (sem0, sem1)
    embufs = (embT_v0, embT_v1)
    gtbufs = (gateT_v0, gateT_v1)
    rvs = [jnp.full((LANES,), r, jnp.int32) for r in range(DGROUP)]

    def issue_faces(g, kp):
        pltpu.async_copy(embT_hbm.at[g], embufs[kp], sem_in)
        pltpu.async_copy(gateT_hbm.at[g], gtbufs[kp], sem_in)

    def wait_faces(g, kp):
        pltpu.make_async_copy(embT_hbm.at[g], embufs[kp], sem_in).wait()
        pltpu.make_async_copy(gateT_hbm.at[g], gtbufs[kp], sem_in).wait()

    issue_faces(wid, 0)

    def pair_body(i, carry):
        for kp in range(2):
            k = i * 2 + kp
            g = k * NUM_WORKERS + wid

            @pl.when(g + NUM_WORKERS < NGROUPS)
            def _():
                issue_faces(g + NUM_WORKERS, 1 - kp)

            @pl.when(g < NGROUPS)
            def _():
                wait_faces(g, kp)
                embT_v = embufs[kp]
                gateT_v = gtbufs[kp]

                def s_body(si, c2):
                    for sp in range(2):
                        s = si * 2 + sp

                        @pl.when(k * SEQ + s >= 2)
                        def _():
                            pltpu.make_async_copy(
                                obufs[sp], out_hbm.at[s, g], sems[sp]).wait()

                        @plsc.parallel_loop(0, BATCH // LANES, unroll=3)
                        def _(bb):
                            tb = bb // 8
                            off = (bb % 8) * LANES
                            pv = packed_v[s, pl.ds(bb * LANES, LANES)]
                            iv = pv & 0xFFF
                            qv = lax.shift_right_logical(pv, 16)
                            it = lax.shift_right_logical(iv, 7)
                            il = iv & 127
                            qt = lax.shift_right_logical(qv, 7)
                            ql = qv & 127
                            es = [plsc.load_gather(embT_v, [it, rvs[r], il])
                                  for r in range(DGROUP)]
                            gs = [plsc.load_gather(gateT_v, [qt, rvs[r], ql])
                                  for r in range(DGROUP)]
                            for r in range(DGROUP):
                                obufs[sp][tb, r, pl.ds(off, LANES)] = es[r] * gs[r]
                        pltpu.async_copy(obufs[sp], out_hbm.at[s, g], sems[sp])
                    return c2

                lax.fori_loop(0, SEQ // 2, s_body, 0)

        return carry

    lax.fori_loop(0, KMAX // 2, pair_body, 0)
    for sp in range(2):
        pltpu.make_async_copy(obufs[sp], out_hbm.at[0, 0], sems[sp]).wait()


def kernel(embeddings, position_embeddings, indices, positions):
    embT = jnp.transpose(embeddings)
    gateT = _transpose_gate(_sigmoid_wide(position_embeddings))
    embT4 = jnp.transpose(
        embT.reshape(NGROUPS, DGROUP, EMB_TILES, 128), (0, 2, 1, 3))
    gateT4 = jnp.transpose(
        gateT.reshape(GGROUPS, DGROUP, POS_TILES, 128), (0, 2, 1, 3))
    packed = jnp.transpose(indices | (positions << 16))
    out5 = _sc_gather_mul(embT4, gateT4, packed)
    return jnp.transpose(out5, (2, 4, 0, 1, 3)).reshape(BATCH, SEQ, DIM)


# speedup vs baseline: 1.0065x; 1.0065x over previous
import functools

import jax
import jax.numpy as jnp
from jax import lax
from jax.experimental import pallas as pl
from jax.experimental.pallas import tpu as pltpu
from jax.experimental.pallas import tpu_sc as plsc

NUM_KMERS = 4096
DIM = 10000
DIM_PAD = 10112
MAX_POSITIONS = 1000
POS_PAD = 1024
BATCH = 1024
SEQ = 20
LANES = 16
DGROUP = 8

NUM_CORES = 2
NUM_SUBCORES = 16
NUM_WORKERS = NUM_CORES * NUM_SUBCORES
NGROUPS = DIM // DGROUP
GGROUPS = DIM_PAD // DGROUP
KMAX = (NGROUPS + NUM_WORKERS - 1) // NUM_WORKERS
EMB_TILES = NUM_KMERS // 128
POS_TILES = POS_PAD // 128
BTILES = BATCH // 128


def _sig_body(in_ref, out_ref):
    out_ref[:, :DIM] = jax.nn.sigmoid(in_ref[...])
    out_ref[:, DIM:] = jnp.ones((in_ref.shape[0], DIM_PAD - DIM), jnp.float32)


def _sigmoid_wide(position_embeddings):
    return pl.pallas_call(
        _sig_body,
        grid=(5,),
        in_specs=[pl.BlockSpec((200, DIM), lambda i: (i, 0))],
        out_specs=pl.BlockSpec((200, DIM_PAD), lambda i: (i, 0)),
        out_shape=jax.ShapeDtypeStruct((POS_PAD, DIM_PAD), jnp.float32),
    )(position_embeddings)


def _t_gate_body(in_ref, out_ref):
    out_ref[...] = in_ref[...].T


def _transpose_gate(gate_wide):
    return pl.pallas_call(
        _t_gate_body,
        grid=(10,),
        in_specs=[pl.BlockSpec((POS_PAD, 1024), lambda c: (0, c))],
        out_specs=pl.BlockSpec((1024, POS_PAD), lambda c: (c, 0)),
        out_shape=jax.ShapeDtypeStruct((DIM_PAD, POS_PAD), jnp.float32),
    )(gate_wide)


_sc_mesh = plsc.VectorSubcoreMesh(core_axis_name="c", subcore_axis_name="s")


@functools.partial(
    pl.kernel,
    mesh=_sc_mesh,
    compiler_params=pltpu.CompilerParams(use_tc_tiling_on_sc=False,
                                         needs_layout_passes=False),
    out_type=jax.ShapeDtypeStruct((SEQ, NGROUPS, BTILES, DGROUP, 128),
                                  jnp.float32),
    scratch_types=[
        pltpu.VMEM((SEQ, BATCH), jnp.int32),
        pltpu.VMEM((EMB_TILES, DGROUP, 128), jnp.float32),
        pltpu.VMEM((EMB_TILES, DGROUP, 128), jnp.float32),
        pltpu.VMEM((POS_TILES, DGROUP, 128), jnp.float32),
        pltpu.VMEM((POS_TILES, DGROUP, 128), jnp.float32),
        pltpu.VMEM((BTILES, DGROUP, 128), jnp.float32),
        pltpu.VMEM((BTILES, DGROUP, 128), jnp.float32),
        pltpu.SemaphoreType.DMA,
        pltpu.SemaphoreType.DMA,
        pltpu.SemaphoreType.DMA,
    ],
)
def _sc_gather_mul(embT_hbm, gateT_hbm, packed_hbm, out_hbm,
                   packed_v, embT_v0, embT_v1, gateT_v0, gateT_v1,
                   ob0, ob1, sem0, sem1, sem_in):
    wid = lax.axis_index("s") * NUM_CORES + lax.axis_index("c")
    pltpu.sync_copy(packed_hbm, packed_v)
    obufs = (ob0, ob1)
    sems = (sem0, sem1)
    embufs = (embT_v0, embT_v1)
    gtbufs = (gateT_v0, gateT_v1)
    rvs = [jnp.full((LANES,), r, jnp.int32) for r in range(DGROUP)]

    def issue_faces(g, kp):
        pltpu.async_copy(embT_hbm.at[g], embufs[kp], sem_in)
        pltpu.async_copy(gateT_hbm.at[g], gtbufs[kp], sem_in)

    def wait_faces(g, kp):
        pltpu.make_async_copy(embT_hbm.at[g], embufs[kp], sem_in).wait()
        pltpu.make_async_copy(gateT_hbm.at[g], gtbufs[kp], sem_in).wait()

    issue_faces(wid, 0)

    def pair_body(i, carry):
        for kp in range(2):
            k = i * 2 + kp
            g = k * NUM_WORKERS + wid

            @pl.when(g + NUM_WORKERS < NGROUPS)
            def _():
                issue_faces(g + NUM_WORKERS, 1 - kp)

            @pl.when(g < NGROUPS)
            def _():
                wait_faces(g, kp)
                embT_v = embufs[kp]
                gateT_v = gtbufs[kp]

                def s_body(si, c2):
                    for sp in range(2):
                        s = si * 2 + sp

                        @pl.when(k * SEQ + s >= 2)
                        def _():
                            pltpu.make_async_copy(
                                obufs[sp], out_hbm.at[s, g], sems[sp]).wait()

                        @plsc.parallel_loop(0, BATCH // LANES, unroll=2)
                        def _(bb):
                            tb = bb // 8
                            off = (bb % 8) * LANES
                            pv = packed_v[s, pl.ds(bb * LANES, LANES)]
                            iv = pv & 0xFFF
                            qv = lax.shift_right_logical(pv, 16)
                            it = lax.shift_right_logical(iv, 7)
                            il = iv & 127
                            qt = lax.shift_right_logical(qv, 7)
                            ql = qv & 127
                            es = [plsc.load_gather(embT_v, [it, rvs[r], il])
                                  for r in range(DGROUP)]
                            gs = [plsc.load_gather(gateT_v, [qt, rvs[r], ql])
                                  for r in range(DGROUP)]
                            for r in range(DGROUP):
                                obufs[sp][tb, r, pl.ds(off, LANES)] = es[r] * gs[r]
                        pltpu.async_copy(obufs[sp], out_hbm.at[s, g], sems[sp])
                    return c2

                lax.fori_loop(0, SEQ // 2, s_body, 0)

        return carry

    lax.fori_loop(0, KMAX // 2, pair_body, 0)
    for sp in range(2):
        pltpu.make_async_copy(obufs[sp], out_hbm.at[0, 0], sems[sp]).wait()


def kernel(embeddings, position_embeddings, indices, positions):
    embT = jnp.transpose(embeddings)
    gateT = _transpose_gate(_sigmoid_wide(position_embeddings))
    embT4 = jnp.transpose(
        embT.reshape(NGROUPS, DGROUP, EMB_TILES, 128), (0, 2, 1, 3))
    gateT4 = jnp.transpose(
        gateT.reshape(GGROUPS, DGROUP, POS_TILES, 128), (0, 2, 1, 3))
    packed = jnp.transpose(indices | (positions << 16))
    out5 = _sc_gather_mul(embT4, gateT4, packed)
    return jnp.transpose(out5, (2, 4, 0, 1, 3)).reshape(BATCH, SEQ, DIM)
